# baseline (speedup 1.0000x reference)
import jax
import jax.numpy as jnp
from jax import lax
from jax.experimental import pallas as pl
from jax.experimental.pallas import tpu as pltpu
from jax.experimental.pallas import tpu_sc as plsc

NUM_CORES = 2
NUM_SUBCORES = 16
LANES = 16
NUM_WORKERS = NUM_CORES * NUM_SUBCORES

B = 16384
NUM_FACTOR = 64
NUM_DIM = NUM_FACTOR // 4
NUM_DEST = 1000000
PANEL = 256
PSHIFT = 8
NUM_PANELS = (NUM_DEST + PANEL - 1) // PANEL
P_PER_W = (NUM_PANELS + NUM_WORKERS - 1) // NUM_WORKERS
NBUF = 3
TAB_STRIDE = NUM_DIM + 1
RING = 8
OUT_PAD = B + 1024
OUT_SH = OUT_PAD
BUCKET_CAP = B + P_PER_W * (LANES - 1) + LANES


def _sc_kernel(ctx_hbm, dest_hbm, tdow_hbm, ttime_hbm, tmonth_hbm, tday_hbm,
               witem_t_hbm, out0_hbm, out1_hbm,
               dest_all, ctx_all, comp_pos, bucketed, hist_v, starts_v,
               cursors_v, tdow_v, ttime_v, tmonth_v, tday_v,
               panel0, panel1, panel2, accbuf, zerobuf, out_sh,
               semp0, semp1, semp2):
  cid = lax.axis_index("c")
  sid = lax.axis_index("s")
  wid = sid * NUM_CORES + cid
  lo = wid * P_PER_W
  hi = jnp.minimum(lo + P_PER_W, NUM_PANELS)
  cnt_p = hi - lo

  lane_iota = lax.iota(jnp.int32, LANES)
  zeros16i = jnp.zeros((LANES,), jnp.int32)

  def sread(ref, i):
    return plsc.load_gather(ref, [jnp.full((LANES,), i, jnp.int32)])[0]

  pltpu.sync_copy(dest_hbm, dest_all)
  pltpu.sync_copy(ctx_hbm, ctx_all)
  pltpu.sync_copy(tdow_hbm, tdow_v)
  pltpu.sync_copy(ttime_hbm, ttime_v)
  pltpu.sync_copy(tmonth_hbm, tmonth_v)
  pltpu.sync_copy(tday_hbm, tday_v)

  for k in range(P_PER_W // LANES + 1):
    hist_v[pl.ds(k * LANES, LANES)] = zeros16i

  zerosf = jnp.zeros((LANES,), jnp.float32)
  for k in range(1024 // LANES):
    zerobuf[pl.ds(k * LANES, LANES)] = zerosf

  @pl.when(sid == 0)
  def _():
    for k in range(OUT_PAD // 1024):
      pltpu.sync_copy(zerobuf, out_sh.at[pl.ds(k * 1024, 1024)])
  plsc.subcore_barrier()

  def p0_body(k, cnt):
    dvec = dest_all[pl.ds(k * LANES, LANES)]
    pvec = lax.shift_right_logical(dvec, PSHIFT)
    m = (pvec >= lo) & (pvec < hi)
    mi = jnp.where(m, 1, 0)
    pref = plsc.cumsum(mi)
    slot = cnt + pref - mi
    slotc = jnp.where(m, slot, B + lane_iota)
    plsc.store_scatter(comp_pos, [slotc], k * LANES + lane_iota, mask=m)
    return cnt + pref[LANES - 1]

  m_total = lax.fori_loop(0, B // LANES, p0_body, jnp.int32(0))
  n_chunks = lax.shift_right_logical(m_total + (LANES - 1), 4)

  def rank_islast(local, valid):
    rank = zeros16i
    is_last = valid
    for j in range(LANES):
      pj = local[j]
      vj = jnp.where(valid, 1, 0)[j] > 0
      eqj = (local == pj) & valid & vj
      rank = rank + jnp.where(eqj & (lane_iota > j), 1, 0)
      is_last = is_last & ~(eqj & (lane_iota < j))
    return rank, is_last

  def load_chunk(j):
    posv = comp_pos[pl.ds(j * LANES, LANES)]
    valid = (j * LANES + lane_iota) < m_total
    posc = jnp.where(valid, posv, 0)
    dv = plsc.load_gather(dest_all, [posc])
    local = lax.shift_right_logical(dv, PSHIFT) - lo
    localc = jnp.where(valid, local, P_PER_W + 8)
    return posv, valid, posc, localc

  def p1_body(j, carry):
    _, valid, _, localc = load_chunk(j)
    rank, is_last = rank_islast(localc, valid)
    old = plsc.load_gather(hist_v, [localc])
    plsc.store_scatter(hist_v, [localc], old + rank + 1,
                       mask=is_last & valid)
    return carry

  lax.fori_loop(0, n_chunks, p1_body, jnp.int32(0))

  run = jnp.int32(0)
  for k in range(P_PER_W // LANES + 1):
    h = hist_v[pl.ds(k * LANES, LANES)]
    ha = (h + (LANES - 1)) & ~(LANES - 1)
    cs = plsc.cumsum(ha)
    st = run + cs - ha
    starts_v[pl.ds(k * LANES, LANES)] = st
    cursors_v[pl.ds(k * LANES, LANES)] = st
    run = run + cs[LANES - 1]

  def p3_body(j, carry):
    posv, valid, _, localc = load_chunk(j)
    rank, is_last = rank_islast(localc, valid)
    cur = plsc.load_gather(cursors_v, [localc])
    slot = cur + rank
    slotc = jnp.where(valid, slot, BUCKET_CAP - LANES + lane_iota)
    plsc.store_scatter(bucketed, [slotc], posv, mask=valid)
    plsc.store_scatter(cursors_v, [localc], slot + 1, mask=is_last & valid)
    return carry

  lax.fori_loop(0, n_chunks, p3_body, jnp.int32(0))

  tabs = (tdow_v, ttime_v, tmonth_v, tday_v)

  def _panel_copies(pg, panel_buf, semp, go):
    off = pl.multiple_of(pg * PANEL, PANEL)
    full = pg < NUM_PANELS - 1
    for tr in range(NUM_FACTOR // 8):
      for width, cond in ((PANEL, full), (PANEL // 2, ~full)):
        cp = pltpu.make_async_copy(
            witem_t_hbm.at[pl.ds(tr * 8, 8), pl.ds(off, width)],
            panel_buf.at[pl.ds(tr * 8, 8), pl.ds(0, width)], semp)

        @pl.when(cond)
        def _(cp=cp, go=go):
          if go:
            cp.start()
          else:
            cp.wait()

  def start_panel(pg, panel_buf, semp):
    _panel_copies(pg, panel_buf, semp, True)

  def wait_panel(pg, panel_buf, semp):
    _panel_copies(pg, panel_buf, semp, False)

  panels = (panel0, panel1, panel2)
  semps = (semp0, semp1, semp2)
  for q in range(NBUF):
    @pl.when(cnt_p >= q + 1)
    def _(q=q):
      start_panel(lo + q, panels[q], semps[q])

  def process_panel(panel_buf, semp, p, rc0):
    pg = lo + p
    wait_panel(pg, panel_buf, semp)
    s0 = sread(starts_v, p)
    np_ = sread(hist_v, p)
    nch = lax.shift_right_logical(np_ + (LANES - 1), 4)

    def chunk_body(c, rc):
      posv = bucketed[pl.ds(s0 + c * LANES, LANES)]
      valid = lane_iota < (np_ - c * LANES)
      posc = jnp.where(valid, posv, 0)
      dv = plsc.load_gather(dest_all, [posc])
      col = dv & (PANEL - 1)
      cx = plsc.load_gather(ctx_all, [posc])
      seg_idx = ((cx & 7) * TAB_STRIDE,
                 (lax.shift_right_logical(cx, 3) & 63) * TAB_STRIDE,
                 (lax.shift_right_logical(cx, 9) & 15) * TAB_STRIDE,
                 (lax.shift_right_logical(cx, 13) & 31) * TAB_STRIDE)
      acc = jnp.zeros((LANES,), jnp.float32)
      for d in range(NUM_FACTOR):
        seg = d >> 4
        dd = d & (NUM_DIM - 1)
        u = plsc.load_gather(tabs[seg], [seg_idx[seg] + dd])
        cc = plsc.load_gather(panel_buf,
                              [jnp.full((LANES,), d, jnp.int32), col])
        acc = acc + u * cc
      bsafe = jnp.where(valid, posv, B + lane_iota)
      accbuf[pl.ds(0, LANES)] = acc
      pltpu.sync_copy(accbuf.at[pl.ds(0, LANES)], out_sh.at[bsafe],
                      add=True)
      return rc + 1

    rc1 = lax.fori_loop(0, nch, chunk_body, rc0)

    @pl.when(p + NBUF < cnt_p)
    def _():
      start_panel(pg + NBUF, panel_buf, semp)
    return rc1

  def sweep_body(p, rc):
    return lax.switch(lax.rem(p, NBUF),
                      [lambda r, q=q: process_panel(panels[q], semps[q], p, r)
                       for q in range(NBUF)],
                      rc)

  lax.fori_loop(0, cnt_p, sweep_body, jnp.int32(0))

  plsc.subcore_barrier()

  @pl.when(sid == 0)
  def _():
    @pl.when(cid == 0)
    def _():
      pltpu.sync_copy(out_sh, out0_hbm)

    @pl.when(cid == 1)
    def _():
      pltpu.sync_copy(out_sh, out1_hbm)


def _pad_tab(w):
  return jnp.pad(w, ((0, 0), (0, TAB_STRIDE - NUM_DIM))).reshape(-1)


@jax.jit
def kernel(dayofweek, time, month, day, destination,
           W_dow, W_time, W_month, W_day, W_item):
  ctx = (dayofweek.astype(jnp.int32)
         | (time.astype(jnp.int32) << 3)
         | (month.astype(jnp.int32) << 9)
         | (day.astype(jnp.int32) << 13))
  dest = destination.astype(jnp.int32)

  mesh = plsc.VectorSubcoreMesh(
      core_axis_name="c", subcore_axis_name="s",
      num_cores=NUM_CORES, num_subcores=NUM_SUBCORES)

  run = pl.kernel(
      _sc_kernel,
      out_type=(jax.ShapeDtypeStruct((OUT_PAD,), jnp.float32),
                jax.ShapeDtypeStruct((OUT_PAD,), jnp.float32)),
      mesh=mesh,
      scratch_types=[
          pltpu.VMEM((B,), jnp.int32),
          pltpu.VMEM((B,), jnp.int32),
          pltpu.VMEM((B + LANES,), jnp.int32),
          pltpu.VMEM((BUCKET_CAP,), jnp.int32),
          pltpu.VMEM((256,), jnp.int32),
          pltpu.VMEM((256,), jnp.int32),
          pltpu.VMEM((256,), jnp.int32),
          pltpu.VMEM((7 * TAB_STRIDE,), jnp.float32),
          pltpu.VMEM((48 * TAB_STRIDE,), jnp.float32),
          pltpu.VMEM((12 * TAB_STRIDE,), jnp.float32),
          pltpu.VMEM((31 * TAB_STRIDE,), jnp.float32),
          pltpu.VMEM((NUM_FACTOR, PANEL), jnp.float32),
          pltpu.VMEM((NUM_FACTOR, PANEL), jnp.float32),
          pltpu.VMEM((NUM_FACTOR, PANEL), jnp.float32),
          pltpu.VMEM((LANES,), jnp.float32),
          pltpu.VMEM((1024,), jnp.float32),
          pltpu.VMEM_SHARED((OUT_SH,), jnp.float32),
          pltpu.SemaphoreType.DMA,
          pltpu.SemaphoreType.DMA,
          pltpu.SemaphoreType.DMA,
      ],
      compiler_params=pltpu.CompilerParams(needs_layout_passes=False),
  )
  out0, out1 = run(ctx, dest,
                   _pad_tab(W_dow), _pad_tab(W_time), _pad_tab(W_month),
                   _pad_tab(W_day), W_item.T)
  return (out0[:B] + out1[:B])


# speedup vs baseline: 1.1148x; 1.1148x over previous
import jax
import jax.numpy as jnp
from jax import lax
from jax.experimental import pallas as pl
from jax.experimental.pallas import tpu as pltpu
from jax.experimental.pallas import tpu_sc as plsc

NUM_CORES = 2
NUM_SUBCORES = 16
LANES = 16
NUM_WORKERS = NUM_CORES * NUM_SUBCORES

B = 16384
NUM_FACTOR = 64
NUM_DIM = NUM_FACTOR // 4
NUM_DEST = 1000000
PANEL = 128
NUM_PANELS = (NUM_DEST + PANEL - 1) // PANEL
P_PER_W = (NUM_PANELS + NUM_WORKERS - 1) // NUM_WORKERS
TAB_STRIDE = NUM_DIM + 1
RING = 8
OUT_PAD = B + 1024
OUT_SH = OUT_PAD
BUCKET_CAP = B + P_PER_W * (LANES - 1) + LANES


def _sc_kernel(ctx_hbm, dest_hbm, tdow_hbm, ttime_hbm, tmonth_hbm, tday_hbm,
               witem_t_hbm, out0_hbm, out1_hbm,
               dest_all, ctx_all, comp_pos, bucketed, hist_v, starts_v,
               cursors_v, tdow_v, ttime_v, tmonth_v, tday_v,
               panel0, panel1, panel2, panel3, panel4, panel5, accbuf, zerobuf,
               out_sh, semp0, semp1, semp2, semp3, semp4, semp5):
  cid = lax.axis_index("c")
  sid = lax.axis_index("s")
  wid = sid * NUM_CORES + cid
  lo = wid * P_PER_W
  hi = jnp.minimum(lo + P_PER_W, NUM_PANELS)
  cnt_p = hi - lo

  lane_iota = lax.iota(jnp.int32, LANES)
  zeros16i = jnp.zeros((LANES,), jnp.int32)

  def sread(ref, i):
    return plsc.load_gather(ref, [jnp.full((LANES,), i, jnp.int32)])[0]

  pltpu.sync_copy(dest_hbm, dest_all)
  pltpu.sync_copy(ctx_hbm, ctx_all)
  pltpu.sync_copy(tdow_hbm, tdow_v)
  pltpu.sync_copy(ttime_hbm, ttime_v)
  pltpu.sync_copy(tmonth_hbm, tmonth_v)
  pltpu.sync_copy(tday_hbm, tday_v)

  for k in range(P_PER_W // LANES + 1):
    hist_v[pl.ds(k * LANES, LANES)] = zeros16i

  zerosf = jnp.zeros((LANES,), jnp.float32)
  for k in range(1024 // LANES):
    zerobuf[pl.ds(k * LANES, LANES)] = zerosf

  @pl.when(sid == 0)
  def _():
    for k in range(OUT_PAD // 1024):
      pltpu.sync_copy(zerobuf, out_sh.at[pl.ds(k * 1024, 1024)])
  plsc.subcore_barrier()

  def p0_body(k, cnt):
    dvec = dest_all[pl.ds(k * LANES, LANES)]
    pvec = lax.shift_right_logical(dvec, 7)
    m = (pvec >= lo) & (pvec < hi)
    mi = jnp.where(m, 1, 0)
    pref = plsc.cumsum(mi)
    slot = cnt + pref - mi
    slotc = jnp.where(m, slot, B + lane_iota)
    plsc.store_scatter(comp_pos, [slotc], k * LANES + lane_iota, mask=m)
    return cnt + pref[LANES - 1]

  m_total = lax.fori_loop(0, B // LANES, p0_body, jnp.int32(0))
  n_chunks = lax.shift_right_logical(m_total + (LANES - 1), 4)

  def rank_islast(local, valid):
    rank = zeros16i
    is_last = valid
    for j in range(LANES):
      pj = local[j]
      vj = jnp.where(valid, 1, 0)[j] > 0
      eqj = (local == pj) & valid & vj
      rank = rank + jnp.where(eqj & (lane_iota > j), 1, 0)
      is_last = is_last & ~(eqj & (lane_iota < j))
    return rank, is_last

  def load_chunk(j):
    posv = comp_pos[pl.ds(j * LANES, LANES)]
    valid = (j * LANES + lane_iota) < m_total
    posc = jnp.where(valid, posv, 0)
    dv = plsc.load_gather(dest_all, [posc])
    local = lax.shift_right_logical(dv, 7) - lo
    localc = jnp.where(valid, local, P_PER_W + 8)
    return posv, valid, posc, localc

  def p1_body(j, carry):
    _, valid, _, localc = load_chunk(j)
    rank, is_last = rank_islast(localc, valid)
    old = plsc.load_gather(hist_v, [localc])
    plsc.store_scatter(hist_v, [localc], old + rank + 1,
                       mask=is_last & valid)
    return carry

  lax.fori_loop(0, n_chunks, p1_body, jnp.int32(0))

  run = jnp.int32(0)
  for k in range(P_PER_W // LANES + 1):
    h = hist_v[pl.ds(k * LANES, LANES)]
    ha = (h + (LANES - 1)) & ~(LANES - 1)
    cs = plsc.cumsum(ha)
    st = run + cs - ha
    starts_v[pl.ds(k * LANES, LANES)] = st
    cursors_v[pl.ds(k * LANES, LANES)] = st
    run = run + cs[LANES - 1]

  def p3_body(j, carry):
    posv, valid, _, localc = load_chunk(j)
    rank, is_last = rank_islast(localc, valid)
    cur = plsc.load_gather(cursors_v, [localc])
    slot = cur + rank
    slotc = jnp.where(valid, slot, BUCKET_CAP - LANES + lane_iota)
    plsc.store_scatter(bucketed, [slotc], posv, mask=valid)
    plsc.store_scatter(cursors_v, [localc], slot + 1, mask=is_last & valid)
    return carry

  lax.fori_loop(0, n_chunks, p3_body, jnp.int32(0))

  tabs = (tdow_v, ttime_v, tmonth_v, tday_v)

  def start_panel(pg, panel_buf, semp):
    off = pl.multiple_of(pg * PANEL, PANEL)
    for tr in range(NUM_FACTOR // 8):
      pltpu.async_copy(
          witem_t_hbm.at[pl.ds(tr * 8, 8), pl.ds(off, PANEL)],
          panel_buf.at[pl.ds(tr * 8, 8), :], semp)

  def wait_panel(pg, panel_buf, semp):
    off = pl.multiple_of(pg * PANEL, PANEL)
    for tr in range(NUM_FACTOR // 8):
      pltpu.make_async_copy(
          witem_t_hbm.at[pl.ds(tr * 8, 8), pl.ds(off, PANEL)],
          panel_buf.at[pl.ds(tr * 8, 8), :], semp).wait()

  panels = (panel0, panel1, panel2, panel3, panel4, panel5)
  semps = (semp0, semp1, semp2, semp3, semp4, semp5)
  NBUF = 6
  for q in range(NBUF):
    @pl.when(cnt_p >= q + 1)
    def _(q=q):
      start_panel(lo + q, panels[q], semps[q])

  def process_panel(panel_buf, semp, p, rc0):
    pg = lo + p
    wait_panel(pg, panel_buf, semp)
    s0 = sread(starts_v, p)
    np_ = sread(hist_v, p)
    nch = lax.shift_right_logical(np_ + (LANES - 1), 4)

    def chunk_body(c, rc):
      posv = bucketed[pl.ds(s0 + c * LANES, LANES)]
      valid = lane_iota < (np_ - c * LANES)
      posc = jnp.where(valid, posv, 0)
      dv = plsc.load_gather(dest_all, [posc])
      col = dv & (PANEL - 1)
      cx = plsc.load_gather(ctx_all, [posc])
      seg_idx = ((cx & 7) * TAB_STRIDE,
                 (lax.shift_right_logical(cx, 3) & 63) * TAB_STRIDE,
                 (lax.shift_right_logical(cx, 9) & 15) * TAB_STRIDE,
                 (lax.shift_right_logical(cx, 13) & 31) * TAB_STRIDE)
      acc = jnp.zeros((LANES,), jnp.float32)
      for d in range(NUM_FACTOR):
        seg = d >> 4
        dd = d & (NUM_DIM - 1)
        u = plsc.load_gather(tabs[seg], [seg_idx[seg] + dd])
        cc = plsc.load_gather(panel_buf,
                              [jnp.full((LANES,), d, jnp.int32), col])
        acc = acc + u * cc
      bsafe = jnp.where(valid, posv, B + lane_iota)
      accbuf[pl.ds(0, LANES)] = acc
      pltpu.sync_copy(accbuf.at[pl.ds(0, LANES)], out_sh.at[bsafe],
                      add=True)
      return rc + 1

    rc1 = lax.fori_loop(0, nch, chunk_body, rc0)

    @pl.when(p + NBUF < cnt_p)
    def _():
      start_panel(pg + NBUF, panel_buf, semp)
    return rc1

  def sweep_body(p, rc):
    return lax.switch(lax.rem(p, NBUF),
                      [lambda r, q=q: process_panel(panels[q], semps[q], p, r)
                       for q in range(NBUF)],
                      rc)

  lax.fori_loop(0, cnt_p, sweep_body, jnp.int32(0))

  plsc.subcore_barrier()

  @pl.when(sid == 0)
  def _():
    @pl.when(cid == 0)
    def _():
      pltpu.sync_copy(out_sh, out0_hbm)

    @pl.when(cid == 1)
    def _():
      pltpu.sync_copy(out_sh, out1_hbm)


def _pad_tab(w):
  return jnp.pad(w, ((0, 0), (0, TAB_STRIDE - NUM_DIM))).reshape(-1)


@jax.jit
def kernel(dayofweek, time, month, day, destination,
           W_dow, W_time, W_month, W_day, W_item):
  ctx = (dayofweek.astype(jnp.int32)
         | (time.astype(jnp.int32) << 3)
         | (month.astype(jnp.int32) << 9)
         | (day.astype(jnp.int32) << 13))
  dest = destination.astype(jnp.int32)

  mesh = plsc.VectorSubcoreMesh(
      core_axis_name="c", subcore_axis_name="s",
      num_cores=NUM_CORES, num_subcores=NUM_SUBCORES)

  run = pl.kernel(
      _sc_kernel,
      out_type=(jax.ShapeDtypeStruct((OUT_PAD,), jnp.float32),
                jax.ShapeDtypeStruct((OUT_PAD,), jnp.float32)),
      mesh=mesh,
      scratch_types=[
          pltpu.VMEM((B,), jnp.int32),
          pltpu.VMEM((B,), jnp.int32),
          pltpu.VMEM((B + LANES,), jnp.int32),
          pltpu.VMEM((BUCKET_CAP,), jnp.int32),
          pltpu.VMEM((256,), jnp.int32),
          pltpu.VMEM((256,), jnp.int32),
          pltpu.VMEM((256,), jnp.int32),
          pltpu.VMEM((7 * TAB_STRIDE,), jnp.float32),
          pltpu.VMEM((48 * TAB_STRIDE,), jnp.float32),
          pltpu.VMEM((12 * TAB_STRIDE,), jnp.float32),
          pltpu.VMEM((31 * TAB_STRIDE,), jnp.float32),
          pltpu.VMEM((NUM_FACTOR, PANEL), jnp.float32),
          pltpu.VMEM((NUM_FACTOR, PANEL), jnp.float32),
          pltpu.VMEM((NUM_FACTOR, PANEL), jnp.float32),
          pltpu.VMEM((NUM_FACTOR, PANEL), jnp.float32),
          pltpu.VMEM((NUM_FACTOR, PANEL), jnp.float32),
          pltpu.VMEM((NUM_FACTOR, PANEL), jnp.float32),
          pltpu.VMEM((LANES,), jnp.float32),
          pltpu.VMEM((1024,), jnp.float32),
          pltpu.VMEM_SHARED((OUT_SH,), jnp.float32),
          pltpu.SemaphoreType.DMA,
          pltpu.SemaphoreType.DMA,
          pltpu.SemaphoreType.DMA,
          pltpu.SemaphoreType.DMA,
          pltpu.SemaphoreType.DMA,
          pltpu.SemaphoreType.DMA,
      ],
      compiler_params=pltpu.CompilerParams(needs_layout_passes=False),
  )
  out0, out1 = run(ctx, dest,
                   _pad_tab(W_dow), _pad_tab(W_time), _pad_tab(W_month),
                   _pad_tab(W_day), W_item.T)
  return (out0[:B] + out1[:B])
